# Initial kernel scaffold; baseline (speedup 1.0000x reference)
import functools

import jax
import jax.numpy as jnp
from jax import lax
from jax.experimental import pallas as pl
from jax.experimental.pallas import tpu as pltpu
from jax.experimental.pallas import tpu_sc as plsc

_N = 10000
_E = 320000
_D = 128
_NC = 2
_NS = 16
_K = 80
_RPS = _N // _NS
_RB = 1250

_MESH = dict(core_axis_name="c", subcore_axis_name="s", num_cores=_NC,
             num_subcores=_NS)


def _sc_degrees(edge_index, ones_blk, zeros_col):
    per_tile = _E // _NS
    n_chunks = per_tile // _K

    @functools.partial(
        pl.kernel,
        out_type=jax.ShapeDtypeStruct((_NC, _N, 1), jnp.float32),
        mesh=plsc.VectorSubcoreMesh(**_MESH),
        scratch_types=[
            pltpu.VMEM((_K,), jnp.int32),
            pltpu.VMEM((_K, 1), jnp.float32),
            pltpu.VMEM_SHARED((_N, 1), jnp.float32),
        ],
    )
    def deg_kernel(edges_hbm, ones_hbm, zeros_hbm, out_hbm, idx_v, ones_v,
                   acc_sh):
        ci = lax.axis_index("c")
        si = lax.axis_index("s")
        pltpu.sync_copy(ones_hbm, ones_v)
        pltpu.sync_copy(zeros_hbm, acc_sh.at[pl.ds(si * _RPS, _RPS)])
        plsc.subcore_barrier()
        base = si * per_tile

        @pl.loop(0, n_chunks)
        def _(c):
            pltpu.sync_copy(edges_hbm.at[ci, pl.ds(base + c * _K, _K)], idx_v)
            pltpu.sync_copy(ones_v, acc_sh.at[idx_v], add=True)

        plsc.subcore_barrier()
        pltpu.sync_copy(acc_sh.at[pl.ds(si * _RPS, _RPS)],
                        out_hbm.at[ci, pl.ds(si * _RPS, _RPS)])

    return deg_kernel(edge_index, ones_blk, zeros_col)


def _sc_agg(g, src, dst, zeros_blk):
    per_tile = _E // (_NC * _NS)
    n_chunks = per_tile // _K

    @functools.partial(
        pl.kernel,
        out_type=jax.ShapeDtypeStruct((_NC, _N, _D), jnp.float32),
        mesh=plsc.VectorSubcoreMesh(**_MESH),
        scratch_types=[
            pltpu.VMEM((_K,), jnp.int32),
            pltpu.VMEM((_K,), jnp.int32),
            pltpu.VMEM((_K, _D), jnp.float32),
            pltpu.VMEM_SHARED((_N, _D), jnp.float32),
        ],
    )
    def agg_kernel(g_hbm, src_hbm, dst_hbm, zeros_hbm, out_hbm, sidx_v,
                   didx_v, rows_v, acc_sh):
        ci = lax.axis_index("c")
        si = lax.axis_index("s")
        pltpu.sync_copy(zeros_hbm, acc_sh.at[pl.ds(si * _RPS, _RPS)])
        plsc.subcore_barrier()
        base = (ci * _NS + si) * per_tile

        @pl.loop(0, n_chunks)
        def _(c):
            off = base + c * _K
            pltpu.sync_copy(src_hbm.at[pl.ds(off, _K)], sidx_v)
            pltpu.sync_copy(dst_hbm.at[pl.ds(off, _K)], didx_v)
            pltpu.sync_copy(g_hbm.at[sidx_v], rows_v)
            pltpu.sync_copy(rows_v, acc_sh.at[didx_v], add=True)

        plsc.subcore_barrier()
        pltpu.sync_copy(acc_sh.at[pl.ds(si * _RPS, _RPS)],
                        out_hbm.at[ci, pl.ds(si * _RPS, _RPS)])

    return agg_kernel(g, src, dst, zeros_blk)


def _norm(deg):
    return jnp.where(deg > 0, lax.rsqrt(jnp.maximum(deg, 1.0)), 0.0)


def _mm(a, b):
    return lax.dot_general(a, b, (((1,), (0,)), ((), ())),
                           precision=lax.Precision.HIGHEST,
                           preferred_element_type=jnp.float32)


def _tc_mm_scale(x, W, degout):
    def body(x_ref, w_ref, d_ref, o_ref):
        o_ref[...] = _mm(x_ref[...], w_ref[...]) * _norm(d_ref[...])

    return pl.pallas_call(
        body,
        grid=(_N // _RB,),
        in_specs=[pl.BlockSpec((_RB, _D), lambda i: (i, 0)),
                  pl.BlockSpec((_D, _D), lambda i: (0, 0)),
                  pl.BlockSpec((_RB, 1), lambda i: (i, 0))],
        out_specs=pl.BlockSpec((_RB, _D), lambda i: (i, 0)),
        out_shape=jax.ShapeDtypeStruct((_N, _D), jnp.float32),
    )(x, W, degout)


def _tc_mid(agg, degin, b1, W2, degout):
    def body(a_ref, di_ref, b_ref, w_ref, do_ref, o_ref):
        s = a_ref[0] + a_ref[1]
        h = jnp.maximum(s * _norm(di_ref[...]) + b_ref[...], 0.0)
        o_ref[...] = _mm(h, w_ref[...]) * _norm(do_ref[...])

    return pl.pallas_call(
        body,
        grid=(_N // _RB,),
        in_specs=[pl.BlockSpec((_NC, _RB, _D), lambda i: (0, i, 0)),
                  pl.BlockSpec((_RB, 1), lambda i: (i, 0)),
                  pl.BlockSpec((1, _D), lambda i: (0, 0)),
                  pl.BlockSpec((_D, _D), lambda i: (0, 0)),
                  pl.BlockSpec((_RB, 1), lambda i: (i, 0))],
        out_specs=pl.BlockSpec((_RB, _D), lambda i: (i, 0)),
        out_shape=jax.ShapeDtypeStruct((_N, _D), jnp.float32),
    )(agg, degin, b1, W2, degout)


def _tc_fin(agg, degin, b2):
    def body(a_ref, di_ref, b_ref, o_ref):
        o_ref[...] = (a_ref[0] + a_ref[1]) * _norm(di_ref[...]) + b_ref[...]

    return pl.pallas_call(
        body,
        grid=(_N // _RB,),
        in_specs=[pl.BlockSpec((_NC, _RB, _D), lambda i: (0, i, 0)),
                  pl.BlockSpec((_RB, 1), lambda i: (i, 0)),
                  pl.BlockSpec((1, _D), lambda i: (0, 0))],
        out_specs=pl.BlockSpec((_RB, _D), lambda i: (i, 0)),
        out_shape=jax.ShapeDtypeStruct((_N, _D), jnp.float32),
    )(agg, degin, b2)


def kernel(x, edge_index, W1, b1, W2, b2):
    src = edge_index[0]
    dst = edge_index[1]
    ones_blk = jnp.ones((_K, 1), jnp.float32)
    zeros_col = jnp.zeros((_RPS, 1), jnp.float32)
    zeros_blk = jnp.zeros((_RPS, _D), jnp.float32)

    deg = _sc_degrees(edge_index, ones_blk, zeros_col)
    degout = deg[0]
    degin = deg[1]

    g1 = _tc_mm_scale(x, W1, degout)
    agg1 = _sc_agg(g1, src, dst, zeros_blk)
    g2 = _tc_mid(agg1, degin, b1.reshape(1, _D), W2, degout)
    agg2 = _sc_agg(g2, src, dst, zeros_blk)
    return _tc_fin(agg2, degin, b2.reshape(1, _D))


# speedup vs baseline: 4.7952x; 4.7952x over previous
import functools

import jax
import jax.numpy as jnp
from jax import lax
from jax.experimental import pallas as pl
from jax.experimental.pallas import tpu as pltpu
from jax.experimental.pallas import tpu_sc as plsc

_N = 10000
_E = 320000
_D = 128
_NC = 2
_NS = 16
_K = 80
_RPS = 1000
_RB = 1000

_MESH = dict(core_axis_name="c", subcore_axis_name="s", num_cores=_NC,
             num_subcores=_NS)


def _sc_degrees(sd):
    per_tile = _E // _NS
    n_chunks = per_tile // _K

    @functools.partial(
        pl.kernel,
        out_type=jax.ShapeDtypeStruct((2 * _N,), jnp.float32),
        mesh=plsc.VectorSubcoreMesh(**_MESH),
        scratch_types=[
            pltpu.VMEM((_K,), jnp.int32),
            pltpu.VMEM((_K,), jnp.float32),
            pltpu.VMEM((_RPS,), jnp.float32),
            pltpu.VMEM_SHARED((_N,), jnp.float32),
        ],
    )
    def deg_kernel(sd_hbm, out_hbm, idx_v, ones_v, zero_v, acc_sh):
        ci = lax.axis_index("c")
        si = lax.axis_index("s")

        @pl.loop(0, _K, step=16)
        def _(i):
            ones_v[pl.ds(i, 16)] = jnp.full((16,), 1.0, jnp.float32)

        @pl.when(si < 10)
        def _():
            @pl.loop(0, _RPS, step=16)
            def _(i):
                zero_v[pl.ds(i, 16)] = jnp.full((16,), 0.0, jnp.float32)

            pltpu.sync_copy(zero_v, acc_sh.at[pl.ds(si * _RPS, _RPS)])

        plsc.subcore_barrier()
        base = ci * _E + si * per_tile

        @pl.loop(0, n_chunks)
        def _(c):
            pltpu.sync_copy(sd_hbm.at[pl.ds(base + c * _K, _K)], idx_v)
            pltpu.sync_copy(ones_v, acc_sh.at[idx_v], add=True)

        plsc.subcore_barrier()

        @pl.when(si < 10)
        def _():
            pltpu.sync_copy(acc_sh.at[pl.ds(si * _RPS, _RPS)], zero_v)
            pltpu.sync_copy(zero_v,
                            out_hbm.at[pl.ds(ci * _N + si * _RPS, _RPS)])

    return deg_kernel(sd)


def _sc_agg(g, src, dst, zeros_blk):
    per_tile = _E // (_NC * _NS)
    n_chunks = per_tile // _K

    @functools.partial(
        pl.kernel,
        out_type=jax.ShapeDtypeStruct((2 * _N, _D), jnp.float32),
        mesh=plsc.VectorSubcoreMesh(**_MESH),
        scratch_types=[
            pltpu.VMEM((_K,), jnp.int32),
            pltpu.VMEM((_K,), jnp.int32),
            pltpu.VMEM((_K, _D), jnp.float32),
            pltpu.VMEM_SHARED((_N, _D), jnp.float32),
        ],
    )
    def agg_kernel(g_hbm, src_hbm, dst_hbm, zeros_hbm, out_hbm, sidx_v,
                   didx_v, rows_v, acc_sh):
        ci = lax.axis_index("c")
        si = lax.axis_index("s")

        @pl.when(si < 10)
        def _():
            pltpu.sync_copy(zeros_hbm, acc_sh.at[pl.ds(si * _RPS, _RPS)])

        plsc.subcore_barrier()
        base = (ci * _NS + si) * per_tile

        @pl.loop(0, n_chunks)
        def _(c):
            off = base + c * _K
            pltpu.sync_copy(src_hbm.at[pl.ds(off, _K)], sidx_v)
            pltpu.sync_copy(dst_hbm.at[pl.ds(off, _K)], didx_v)
            pltpu.sync_copy(g_hbm.at[sidx_v], rows_v)
            pltpu.sync_copy(rows_v, acc_sh.at[didx_v], add=True)

        plsc.subcore_barrier()

        @pl.when(si < 10)
        def _():
            pltpu.sync_copy(acc_sh.at[pl.ds(si * _RPS, _RPS)],
                            out_hbm.at[pl.ds(ci * _N + si * _RPS, _RPS)])

    return agg_kernel(g, src, dst, zeros_blk)


def _norm(deg):
    return jnp.where(deg > 0, lax.rsqrt(jnp.maximum(deg, 1.0)), 0.0)


def _mm(a, b):
    return lax.dot_general(a, b, (((1,), (0,)), ((), ())),
                           precision=lax.Precision.HIGHEST,
                           preferred_element_type=jnp.float32)


def _tc_mm_scale(x, W, degout):
    def body(x_ref, w_ref, d_ref, o_ref):
        o_ref[...] = _mm(x_ref[...], w_ref[...]) * _norm(d_ref[...])

    return pl.pallas_call(
        body,
        grid=(_N // _RB,),
        in_specs=[pl.BlockSpec((_RB, _D), lambda i: (i, 0)),
                  pl.BlockSpec((_D, _D), lambda i: (0, 0)),
                  pl.BlockSpec((_RB, 1), lambda i: (i, 0))],
        out_specs=pl.BlockSpec((_RB, _D), lambda i: (i, 0)),
        out_shape=jax.ShapeDtypeStruct((_N, _D), jnp.float32),
    )(x, W, degout)


def _tc_mid(agg, degin, b1, W2, degout):
    def body(a_ref, di_ref, b_ref, w_ref, do_ref, o_ref):
        s = a_ref[0] + a_ref[1]
        h = jnp.maximum(s * _norm(di_ref[...]) + b_ref[...], 0.0)
        o_ref[...] = _mm(h, w_ref[...]) * _norm(do_ref[...])

    return pl.pallas_call(
        body,
        grid=(_N // _RB,),
        in_specs=[pl.BlockSpec((_NC, _RB, _D), lambda i: (0, i, 0)),
                  pl.BlockSpec((_RB, 1), lambda i: (i, 0)),
                  pl.BlockSpec((1, _D), lambda i: (0, 0)),
                  pl.BlockSpec((_D, _D), lambda i: (0, 0)),
                  pl.BlockSpec((_RB, 1), lambda i: (i, 0))],
        out_specs=pl.BlockSpec((_RB, _D), lambda i: (i, 0)),
        out_shape=jax.ShapeDtypeStruct((_N, _D), jnp.float32),
    )(agg, degin, b1, W2, degout)


def _tc_fin(agg, degin, b2):
    def body(a_ref, di_ref, b_ref, o_ref):
        o_ref[...] = (a_ref[0] + a_ref[1]) * _norm(di_ref[...]) + b_ref[...]

    return pl.pallas_call(
        body,
        grid=(_N // _RB,),
        in_specs=[pl.BlockSpec((_NC, _RB, _D), lambda i: (0, i, 0)),
                  pl.BlockSpec((_RB, 1), lambda i: (i, 0)),
                  pl.BlockSpec((1, _D), lambda i: (0, 0))],
        out_specs=pl.BlockSpec((_RB, _D), lambda i: (i, 0)),
        out_shape=jax.ShapeDtypeStruct((_N, _D), jnp.float32),
    )(agg, degin, b2)


def kernel(x, edge_index, W1, b1, W2, b2):
    src = edge_index[0]
    dst = edge_index[1]
    sd = jnp.concatenate([src, dst])
    zeros_blk = jnp.zeros((_RPS, _D), jnp.float32)

    deg = _sc_degrees(sd).reshape(_NC, _N, 1)
    degout = deg[0]
    degin = deg[1]

    g1 = _tc_mm_scale(x, W1, degout)
    agg1 = _sc_agg(g1, src, dst, zeros_blk).reshape(_NC, _N, _D)
    g2 = _tc_mid(agg1, degin, b1.reshape(1, _D), W2, degout)
    agg2 = _sc_agg(g2, src, dst, zeros_blk).reshape(_NC, _N, _D)
    return _tc_fin(agg2, degin, b2.reshape(1, _D))
